# baseline (speedup 1.0000x reference)
import functools

import jax
import jax.numpy as jnp
from jax import lax
from jax.experimental import pallas as pl
from jax.experimental.pallas import tpu as pltpu
from jax.experimental.pallas import tpu_sc as plsc

SEQ = 200
BATCH = 4096
DIM = 64
VOCAB = 1000000

NC = 2
NS = 16
NW = NC * NS
BPW = BATCH // NW
C1 = 128
C2 = SEQ - C1
LANES = 16
NJ = DIM // LANES


def _pool_body(xt_hbm, table_hbm, out_hbm, idx_v, rows0, rows1, out_v,
               sem0, sem1):
    wid = lax.axis_index("s") * NC + lax.axis_index("c")
    base = wid * BPW

    pltpu.sync_copy(xt_hbm.at[pl.ds(base, BPW)], idx_v)

    def fire(b, rows, sem):
        pltpu.async_copy(table_hbm.at[idx_v.at[b, pl.ds(0, C1)]],
                         rows.at[pl.ds(0, C1)], sem)
        pltpu.async_copy(table_hbm.at[idx_v.at[b, pl.ds(C1, C2)]],
                         rows.at[pl.ds(C1, C2)], sem)

    def drain(b, rows, sem):
        pltpu.make_async_copy(table_hbm.at[idx_v.at[b, pl.ds(0, C1)]],
                              rows.at[pl.ds(0, C1)], sem).wait()
        pltpu.make_async_copy(table_hbm.at[idx_v.at[b, pl.ds(C1, C2)]],
                              rows.at[pl.ds(C1, C2)], sem).wait()

    def reduce_rows(b, rows):
        def red(s, accs):
            return tuple(
                jnp.maximum(a, rows[s, pl.ds(j * LANES, LANES)])
                for j, a in enumerate(accs))
        init = tuple(
            jnp.full((LANES,), -jnp.inf, jnp.float32) for _ in range(NJ))
        accs = lax.fori_loop(0, SEQ, red, init, unroll=8)
        for j in range(NJ):
            out_v[b, pl.ds(j * LANES, LANES)] = accs[j]

    fire(0, rows0, sem0)
    fire(1, rows1, sem1)

    def step(i, carry):
        for ph, (rows, sem) in enumerate(((rows0, sem0), (rows1, sem1))):
            b = 2 * i + ph
            drain(b, rows, sem)
            reduce_rows(b, rows)
            nb = b + 2

            @pl.when(nb < BPW)
            def _():
                fire(nb, rows, sem)
        return carry

    lax.fori_loop(0, BPW // 2, step, 0)

    pltpu.sync_copy(out_v, out_hbm.at[pl.ds(base, BPW)])


def kernel(x, table):
    xt = x.T

    mesh = plsc.VectorSubcoreMesh(
        core_axis_name="c", subcore_axis_name="s",
        num_cores=NC, num_subcores=NS)

    pool = functools.partial(
        pl.kernel,
        out_type=jax.ShapeDtypeStruct((BATCH, DIM), jnp.float32),
        mesh=mesh,
        compiler_params=pltpu.CompilerParams(use_tc_tiling_on_sc=False),
        scratch_types=[
            pltpu.VMEM((BPW, SEQ), jnp.int32),
            pltpu.VMEM((SEQ, DIM), jnp.float32),
            pltpu.VMEM((SEQ, DIM), jnp.float32),
            pltpu.VMEM((BPW, DIM), jnp.float32),
            pltpu.SemaphoreType.DMA,
            pltpu.SemaphoreType.DMA,
        ],
    )(_pool_body)

    return pool(xt, table)
